# Initial kernel scaffold; baseline (speedup 1.0000x reference)
import jax
import jax.numpy as jnp
from jax.experimental import pallas as pl


def kernel(X_w, X_d, deep_tables, wide_table):
    raise NotImplementedError("write your pallas kernel here")


# speedup vs baseline: 1.0883x; 1.0883x over previous
import functools

import jax
import jax.numpy as jnp
from jax import lax
from jax.experimental import pallas as pl
from jax.experimental.pallas import tpu as pltpu
from jax.experimental.pallas import tpu_sc as plsc

N_FIELDS = 26
VOCAB = 100000
EMB = 32
BATCH = 4096
HIST = 20
WIDE_DIM = N_FIELDS * VOCAB

NC = 2
NS = 16
L = 16
NW = NC * NS

BC = 128
NBCHUNK = BATCH // BC
TOTAL_CHUNKS = N_FIELDS * NBCHUNK
CHUNKS_PER_W = TOTAL_CHUNKS // NW
IDX_PER_CHUNK = BC * HIST
GATHER_W = 128
NGATHER = IDX_PER_CHUNK // GATHER_W

WIDE_PER_W = BATCH * N_FIELDS // NW
WGATHER = WIDE_PER_W // GATHER_W
WHALF = WIDE_PER_W // 2
WROWS = (WIDE_DIM + 1 + 15) // 16

_mesh = plsc.VectorSubcoreMesh(core_axis_name="c", subcore_axis_name="s")


@functools.partial(
    pl.kernel,
    out_type=(
        jax.ShapeDtypeStruct((BATCH, N_FIELDS, EMB), jnp.float32),
        jax.ShapeDtypeStruct((NW, L), jnp.float32),
    ),
    mesh=_mesh,
    compiler_params=pltpu.CompilerParams(
        use_tc_tiling_on_sc=False, needs_layout_passes=False),
    scratch_types=[
        pltpu.VMEM((NGATHER, GATHER_W), jnp.int32),
        pltpu.VMEM((IDX_PER_CHUNK, EMB), jnp.float32),
        pltpu.VMEM((BC, 1, EMB), jnp.float32),
        pltpu.VMEM((WGATHER, GATHER_W), jnp.int32),
        pltpu.VMEM((WGATHER, GATHER_W), jnp.int32),
        pltpu.VMEM((WHALF, 16), jnp.float32),
        pltpu.VMEM((L,), jnp.float32),
        pltpu.SemaphoreType.DMA,
    ],
)
def _sc_embed(xd_hbm, xw_hbm, deep_hbm, wide_hbm, out_hbm, wpart_hbm,
              idx_v, rows_v, outc_v, widx_v, wrow_v, wval_v, wacc_v, sem):
    wid = lax.axis_index("s") * NC + lax.axis_index("c")

    @pl.loop(0, CHUNKS_PER_W)
    def _deep_chunk(c):
        gc = wid * CHUNKS_PER_W + c
        f = gc // NBCHUNK
        bc = gc % NBCHUNK
        b0 = bc * BC
        pltpu.sync_copy(xd_hbm.at[f, pl.ds(bc * HIST, NGATHER)], idx_v)
        copies = []
        for j in range(NGATHER):
            copies.append(pltpu.async_copy(
                deep_hbm.at[f].at[idx_v.at[j]],
                rows_v.at[pl.ds(j * GATHER_W, GATHER_W)],
                sem,
            ))
        for cp in copies:
            cp.wait()

        @pl.loop(0, BC)
        def _pool(i):
            j0 = i * HIST
            a0 = rows_v[j0, pl.ds(0, L)]
            a1 = rows_v[j0, pl.ds(L, L)]
            for h in range(1, HIST):
                a0 = a0 + rows_v[j0 + h, pl.ds(0, L)]
                a1 = a1 + rows_v[j0 + h, pl.ds(L, L)]
            outc_v[i, 0, pl.ds(0, L)] = a0
            outc_v[i, 0, pl.ds(L, L)] = a1

        pltpu.sync_copy(outc_v, out_hbm.at[pl.ds(b0, BC), pl.ds(f, 1)])

    pltpu.sync_copy(xw_hbm.at[pl.ds(wid * WGATHER, WGATHER)], widx_v)

    @pl.loop(0, WGATHER)
    def _rowids(r):
        for g in range(GATHER_W // L):
            wrow_v[r, pl.ds(g * L, L)] = widx_v[r, pl.ds(g * L, L)] >> 4

    lane = lax.iota(jnp.int32, L)
    wacc_v[...] = jnp.zeros((L,), jnp.float32)
    for half in range(2):
        wcopies = []
        for j in range(WGATHER // 2):
            wcopies.append(pltpu.async_copy(
                wide_hbm.at[wrow_v.at[half * (WGATHER // 2) + j]],
                wval_v.at[pl.ds(j * GATHER_W, GATHER_W)],
                sem,
            ))
        for cp in wcopies:
            cp.wait()

        @pl.loop(0, WHALF // L)
        def _wsum(k):
            r = half * (WGATHER // 2) + (k >> 3)
            c = (k & 7) * L
            lanes = widx_v[r, pl.ds(c, L)] & 15
            pos = lane + k * L
            wacc_v[...] = wacc_v[...] + plsc.load_gather(wval_v, [pos, lanes])

    pltpu.sync_copy(wacc_v, wpart_hbm.at[wid])


def _finish_body(p_ref, o_ref):
    o_ref[0, 0] = jax.nn.sigmoid(jnp.sum(p_ref[...]))


_finish = pl.pallas_call(
    _finish_body,
    out_specs=pl.BlockSpec(memory_space=pltpu.SMEM),
    out_shape=jax.ShapeDtypeStruct((1, 1), jnp.float32),
)


def kernel(X_w, X_d, deep_tables, wide_table):
    xd = X_d.reshape(N_FIELDS, BATCH * HIST // GATHER_W, GATHER_W)
    xw = X_w.reshape(BATCH * N_FIELDS // GATHER_W, GATHER_W)
    wt16 = jnp.pad(wide_table, ((0, WROWS * 16 - (WIDE_DIM + 1)), (0, 0)))
    wt16 = wt16.reshape(WROWS, 16)
    x_deep3, wpart = _sc_embed(xd, xw, deep_tables, wt16)
    x_deep = x_deep3.reshape(BATCH, N_FIELDS * EMB)
    out = _finish(wpart)[0, 0]
    return (x_deep, out)


# speedup vs baseline: 4.5052x; 4.1397x over previous
import functools

import jax
import jax.numpy as jnp
from jax import lax
from jax.experimental import pallas as pl
from jax.experimental.pallas import tpu as pltpu
from jax.experimental.pallas import tpu_sc as plsc

N_FIELDS = 26
VOCAB = 100000
EMB = 32
BATCH = 4096
HIST = 20
WIDE_DIM = N_FIELDS * VOCAB

NC = 2
NS = 16
L = 16
NW = NC * NS

VPAD = 782 * 128
TASKS = N_FIELDS * EMB
TASKS_PER_W = TASKS // NW
BC = 128
NCHUNK = BATCH // BC

WROWS = 20320
WIDE_PER_W = BATCH * N_FIELDS // NW
WBURST = 32
NWBURST = WIDE_PER_W // WBURST

_mesh = plsc.VectorSubcoreMesh(core_axis_name="c", subcore_axis_name="s")


@functools.partial(
    pl.kernel,
    out_type=(
        jax.ShapeDtypeStruct((N_FIELDS, EMB, BATCH), jnp.float32),
        jax.ShapeDtypeStruct((NW, L), jnp.float32),
    ),
    mesh=_mesh,
    compiler_params=pltpu.CompilerParams(
        use_tc_tiling_on_sc=False, needs_layout_passes=False),
    scratch_types=[
        pltpu.VMEM((782, 128), jnp.float32),
        pltpu.VMEM((HIST, BC), jnp.int32),
        pltpu.VMEM((HIST, BC), jnp.int32),
        pltpu.VMEM((BATCH,), jnp.float32),
        pltpu.VMEM((NWBURST // 4, 128), jnp.int32),
        pltpu.VMEM((NWBURST // 4, 128), jnp.int32),
        pltpu.VMEM((WBURST, 128), jnp.float32),
        pltpu.VMEM((WBURST, 128), jnp.float32),
        pltpu.VMEM((L,), jnp.float32),
        pltpu.SemaphoreType.DMA,
        pltpu.SemaphoreType.DMA,
        pltpu.SemaphoreType.DMA,
    ],
)
def _sc_embed(xd_hbm, xw_hbm, deep_hbm, wide_hbm, out_hbm, wpart_hbm,
              vline_v, idx0_v, idx1_v, outc_v, widx_v, wrow_v,
              wv0_v, wv1_v, wacc_v, sem_i, sem_o, sem_w):
    wid = lax.axis_index("s") * NC + lax.axis_index("c")
    idx_slots = (idx0_v, idx1_v)

    @pl.loop(0, TASKS_PER_W)
    def _task(t):
        gt = wid * TASKS_PER_W + t
        f = gt >> 5
        e = gt & 31
        pltpu.sync_copy(deep_hbm.at[f, e], vline_v)
        pltpu.async_copy(xd_hbm.at[f, :, pl.ds(0, BC)], idx0_v, sem_i)
        @pl.when(t > 0)
        def _():
            pltpu.make_async_copy(outc_v, out_hbm.at[f, e], sem_o).wait()

        @pl.loop(0, NCHUNK // 2)
        def _chunkpair(p):
            for s in range(2):
                c = p * 2 + s
                pltpu.make_async_copy(
                    xd_hbm.at[f, :, pl.ds(0, BC)], idx_slots[s], sem_i,
                ).wait()

                @pl.when(c + 1 < NCHUNK)
                def _():
                    pltpu.async_copy(
                        xd_hbm.at[f, :, pl.ds((c + 1) * BC, BC)],
                        idx_slots[1 - s], sem_i)

                idx_v = idx_slots[s]
                for g in range(BC // L):
                    a0 = jnp.zeros((L,), jnp.float32)
                    a1 = jnp.zeros((L,), jnp.float32)
                    for h in range(HIST):
                        iv = idx_v[h, pl.ds(g * L, L)]
                        val = plsc.load_gather(vline_v, [iv >> 7, iv & 127])
                        if h & 1:
                            a1 = a1 + val
                        else:
                            a0 = a0 + val
                    outc_v[pl.ds(c * BC + g * L, L)] = a0 + a1

        pltpu.async_copy(outc_v, out_hbm.at[f, e], sem_o)

    pltpu.make_async_copy(outc_v, out_hbm.at[0, 0], sem_o).wait()

    pltpu.sync_copy(xw_hbm.at[pl.ds(wid * (NWBURST // 4), NWBURST // 4)],
                    widx_v)

    @pl.loop(0, NWBURST // 4)
    def _rowids(r):
        for g in range(128 // L):
            wrow_v[r, pl.ds(g * L, L)] = widx_v[r, pl.ds(g * L, L)] >> 7

    lane = lax.iota(jnp.int32, L)
    wacc_v[...] = jnp.zeros((L,), jnp.float32)
    wv_slots = (wv0_v, wv1_v)
    pltpu.async_copy(wide_hbm.at[wrow_v.at[0, pl.ds(0, WBURST)]], wv0_v,
                     sem_w)

    @pl.loop(0, NWBURST // 2)
    def _wpair(p):
        for s in range(2):
            b = p * 2 + s
            r = b >> 2
            q = (b & 3) * WBURST
            pltpu.make_async_copy(
                wide_hbm.at[wrow_v.at[0, pl.ds(0, WBURST)]], wv_slots[s],
                sem_w).wait()

            @pl.when(b + 1 < NWBURST)
            def _():
                b1 = b + 1
                pltpu.async_copy(
                    wide_hbm.at[wrow_v.at[b1 >> 2,
                                          pl.ds((b1 & 3) * WBURST, WBURST)]],
                    wv_slots[1 - s], sem_w)

            acc = wacc_v[...]
            for g in range(WBURST // L):
                cols = widx_v[r, pl.ds(q + g * L, L)] & 127
                acc = acc + plsc.load_gather(wv_slots[s],
                                             [lane + g * L, cols])
            wacc_v[...] = acc

    pltpu.sync_copy(wacc_v, wpart_hbm.at[wid])


def _finish_body(p_ref, o_ref):
    o_ref[0, 0] = jax.nn.sigmoid(jnp.sum(p_ref[...]))


_finish = pl.pallas_call(
    _finish_body,
    out_specs=pl.BlockSpec(memory_space=pltpu.SMEM),
    out_shape=jax.ShapeDtypeStruct((1, 1), jnp.float32),
)


def kernel(X_w, X_d, deep_tables, wide_table):
    dp4 = jnp.pad(deep_tables.transpose(0, 2, 1),
                  ((0, 0), (0, 0), (0, VPAD - (VOCAB + 1))))
    dp4 = dp4.reshape(N_FIELDS, EMB, VPAD // 128, 128)
    xdt = X_d.transpose(0, 2, 1)
    xw = X_w.reshape(BATCH * N_FIELDS // 128, 128)
    wt = jnp.pad(wide_table, ((0, WROWS * 128 - (WIDE_DIM + 1)), (0, 0)))
    wt = wt.reshape(WROWS, 128)
    out3, wpart = _sc_embed(xdt, xw, dp4, wt)
    x_deep = out3.transpose(2, 0, 1).reshape(BATCH, N_FIELDS * EMB)
    out = _finish(wpart)[0, 0]
    return (x_deep, out)
